# Initial kernel scaffold; baseline (speedup 1.0000x reference)
import jax
import jax.numpy as jnp
from jax.experimental import pallas as pl


def kernel(x, edge_index, batch, W1, att_src1, att_dst1, b1, W2, att_src2, att_dst2, b2, lin_w, lin_b):
    raise NotImplementedError("write your pallas kernel here")


# speedup vs baseline: 41.1621x; 41.1621x over previous
import functools

import jax
import jax.numpy as jnp
from jax import lax
from jax.experimental import pallas as pl
from jax.experimental.pallas import tpu as pltpu
from jax.experimental.pallas import tpu_sc as plsc

N = 10000
D = 128
F1 = 64
H1 = 8
F2 = 32
G = 128
E = 320000
ET = E + N

NC, NS = 2, 16
NW = NC * NS
K = 128
BPT = -(-ET // (NW * K))
EPAD = BPT * NW * K
NPAD = 10240
RPT = NPAD // NS
NTBL = 10016

BR = 128
GR = -(-N // BR)

def _dense1_body(x_ref, w_ref, a_ref, h_ref, att_ref):
    h = jnp.dot(x_ref[...], w_ref[...], preferred_element_type=jnp.float32)
    h_ref[...] = h
    att_ref[...] = jnp.dot(h, a_ref[...], preferred_element_type=jnp.float32)


_dense1 = pl.pallas_call(
    _dense1_body,
    grid=(GR,),
    in_specs=[
        pl.BlockSpec((BR, D), lambda i: (i, 0)),
        pl.BlockSpec((D, F1), lambda i: (0, 0)),
        pl.BlockSpec((F1, 16), lambda i: (0, 0)),
    ],
    out_specs=[
        pl.BlockSpec((BR, F1), lambda i: (i, 0)),
        pl.BlockSpec((BR, 16), lambda i: (i, 0)),
    ],
    out_shape=[
        jax.ShapeDtypeStruct((N, F1), jnp.float32),
        jax.ShapeDtypeStruct((N, 16), jnp.float32),
    ],
)


def _edge1_body(table, adst_t, srcr, dstr, out,
                src_blk, dst_blk, grows, msg, arow, acc):
    cid = lax.axis_index("c")
    sid = lax.axis_index("s")
    wid = sid * NC + cid

    pltpu.sync_copy(srcr.at[wid], src_blk)
    pltpu.sync_copy(dstr.at[wid], dst_blk)

    zero16 = jnp.zeros((16,), jnp.float32)

    def _zrow(i, _):
        for c0 in range(4):
            msg[i, pl.ds(c0 * 16, 16)] = zero16
        msg[i, pl.ds(72 - 16, 16)] = zero16
        return 0
    lax.fori_loop(0, K, _zrow, 0)

    for q in range(RPT // K):
        pltpu.sync_copy(msg, acc.at[pl.ds(sid * RPT + q * K, K)])
    plsc.subcore_barrier()

    lanes = lax.iota(jnp.int32, 16)

    def _blk(b, _):
        pltpu.sync_copy(table.at[src_blk.at[b]], grows)
        pltpu.sync_copy(adst_t.at[dst_blk.at[b]], arow)

        def _grp(j, _):
            base = j * 16
            ei = lanes + base
            for h in range(H1):
                wcol = jnp.full((16,), F1 + h, jnp.int32)
                asr = plsc.load_gather(grows, [ei, wcol])
                ads = plsc.load_gather(arow, [ei, jnp.full((16,), h, jnp.int32)])
                a = asr + ads
                a = jnp.where(a >= 0.0, a, a * 0.2)
                w = jnp.exp(a)
                plsc.store_scatter(msg, [ei, wcol], w)
                for c0 in range(8):
                    fc = jnp.full((16,), h * 8 + c0, jnp.int32)
                    hv = plsc.load_gather(grows, [ei, fc])
                    plsc.store_scatter(msg, [ei, fc], hv * w)
            return 0
        lax.fori_loop(0, K // 16, _grp, 0)
        pltpu.sync_copy(msg, acc.at[dst_blk.at[b]], add=True)
        return 0
    lax.fori_loop(0, BPT, _blk, 0)

    plsc.subcore_barrier()
    pltpu.sync_copy(acc.at[pl.ds(sid * RPT, RPT)],
                    out.at[cid].at[pl.ds(sid * RPT, RPT)])


@functools.cache
def _edge1():
    mesh = plsc.VectorSubcoreMesh(
        core_axis_name="c", subcore_axis_name="s",
        num_cores=NC, num_subcores=NS)
    return pl.kernel(
        _edge1_body,
        out_type=jax.ShapeDtypeStruct((NC, NPAD, 72), jnp.float32),
        mesh=mesh,
        compiler_params=pltpu.CompilerParams(
            needs_layout_passes=False, use_tc_tiling_on_sc=False),
        scratch_types=[
            pltpu.VMEM((BPT, K), jnp.int32),
            pltpu.VMEM((BPT, K), jnp.int32),
            pltpu.VMEM((K, 72), jnp.float32),
            pltpu.VMEM((K, 72), jnp.float32),
            pltpu.VMEM((K, 8), jnp.float32),
            pltpu.VMEM_SHARED((NPAD, 72), jnp.float32),
        ],
    )


def _dense2_body(s_ref, d_ref, b1_ref, exp8_ref, w2_ref, a2_ref, h2_ref, att2_ref):
    s = s_ref[0] + s_ref[1]
    den = d_ref[0] + d_ref[1]
    rec = 1.0 / (den + 1e-16)
    recx = jnp.dot(rec, exp8_ref[...], preferred_element_type=jnp.float32)
    h1 = s * recx + b1_ref[...]
    act = jnp.where(h1 > 0.0, h1, jnp.exp(jnp.minimum(h1, 0.0)) - 1.0)
    h2 = jnp.dot(act, w2_ref[...], preferred_element_type=jnp.float32)
    h2_ref[...] = h2
    att2_ref[...] = jnp.dot(h2, a2_ref[...], preferred_element_type=jnp.float32)


_dense2 = pl.pallas_call(
    _dense2_body,
    grid=(GR,),
    in_specs=[
        pl.BlockSpec((NC, BR, F1), lambda i: (0, i, 0)),
        pl.BlockSpec((NC, BR, 8), lambda i: (0, i, 0)),
        pl.BlockSpec((1, F1), lambda i: (0, 0)),
        pl.BlockSpec((8, F1), lambda i: (0, 0)),
        pl.BlockSpec((F1, F2), lambda i: (0, 0)),
        pl.BlockSpec((F2, 2), lambda i: (0, 0)),
    ],
    out_specs=[
        pl.BlockSpec((BR, F2), lambda i: (i, 0)),
        pl.BlockSpec((BR, 2), lambda i: (i, 0)),
    ],
    out_shape=[
        jax.ShapeDtypeStruct((N, F2), jnp.float32),
        jax.ShapeDtypeStruct((N, 2), jnp.float32),
    ],
)


def _edge2_body(table, asrc_t, adst_t, srcr, dstr, out,
                src_blk, dst_blk, grows, msg, asrc_l, adst_l, acc):
    cid = lax.axis_index("c")
    sid = lax.axis_index("s")
    wid = sid * NC + cid

    pltpu.sync_copy(srcr.at[wid], src_blk)
    pltpu.sync_copy(dstr.at[wid], dst_blk)
    pltpu.sync_copy(asrc_t, asrc_l)
    pltpu.sync_copy(adst_t, adst_l)

    zero16 = jnp.zeros((16,), jnp.float32)

    def _zrow(i, _):
        for c0 in range(2):
            msg[i, pl.ds(c0 * 16, 16)] = zero16
        msg[i, pl.ds(40 - 16, 16)] = zero16
        return 0
    lax.fori_loop(0, K, _zrow, 0)

    for q in range(RPT // K):
        pltpu.sync_copy(msg, acc.at[pl.ds(sid * RPT + q * K, K)])
    plsc.subcore_barrier()

    lanes = lax.iota(jnp.int32, 16)

    def _blk(b, _):
        pltpu.sync_copy(table.at[src_blk.at[b]], grows)

        def _grp(j, _):
            base = j * 16
            ei = lanes + base
            src16 = src_blk[b, pl.ds(base, 16)]
            dst16 = dst_blk[b, pl.ds(base, 16)]
            asr = plsc.load_gather(asrc_l, [src16])
            ads = plsc.load_gather(adst_l, [dst16])
            a = asr + ads
            a = jnp.where(a >= 0.0, a, a * 0.2)
            w = jnp.exp(a)
            plsc.store_scatter(msg, [ei, jnp.full((16,), F2, jnp.int32)], w)
            for c0 in range(F2):
                fc = jnp.full((16,), c0, jnp.int32)
                hv = plsc.load_gather(grows, [ei, fc])
                plsc.store_scatter(msg, [ei, fc], hv * w)
            return 0
        lax.fori_loop(0, K // 16, _grp, 0)
        pltpu.sync_copy(msg, acc.at[dst_blk.at[b]], add=True)
        return 0
    lax.fori_loop(0, BPT, _blk, 0)

    plsc.subcore_barrier()
    pltpu.sync_copy(acc.at[pl.ds(sid * RPT, RPT)],
                    out.at[cid].at[pl.ds(sid * RPT, RPT)])


@functools.cache
def _edge2():
    mesh = plsc.VectorSubcoreMesh(
        core_axis_name="c", subcore_axis_name="s",
        num_cores=NC, num_subcores=NS)
    return pl.kernel(
        _edge2_body,
        out_type=jax.ShapeDtypeStruct((NC, NPAD, 40), jnp.float32),
        mesh=mesh,
        compiler_params=pltpu.CompilerParams(
            needs_layout_passes=False, use_tc_tiling_on_sc=False),
        scratch_types=[
            pltpu.VMEM((BPT, K), jnp.int32),
            pltpu.VMEM((BPT, K), jnp.int32),
            pltpu.VMEM((K, F2), jnp.float32),
            pltpu.VMEM((K, 40), jnp.float32),
            pltpu.VMEM((NTBL,), jnp.float32),
            pltpu.VMEM((NTBL,), jnp.float32),
            pltpu.VMEM_SHARED((NPAD, 40), jnp.float32),
        ],
    )


def _pool_body(s2_ref, d2_ref, bt_ref, b2_ref, lw_ref, lb_ref, out_ref,
               acc_sum, acc_cnt):
    i = pl.program_id(0)

    @pl.when(i == 0)
    def _():
        acc_sum[...] = jnp.zeros((G, F2), jnp.float32)
        acc_cnt[...] = jnp.zeros((G, 1), jnp.float32)

    s = s2_ref[0] + s2_ref[1]
    den = d2_ref[0] + d2_ref[1]
    h = s / (den + 1e-16) + b2_ref[...]
    colidx = i * BR + lax.broadcasted_iota(jnp.int32, (1, BR), 1)
    validt = colidx < N
    gids = lax.broadcasted_iota(jnp.int32, (G, BR), 0)
    oht = jnp.where((bt_ref[...] == gids) & validt, 1.0, 0.0)
    acc_sum[...] += jnp.dot(oht, h, preferred_element_type=jnp.float32)
    acc_cnt[...] += jnp.sum(oht, axis=1, keepdims=True)

    @pl.when(i == GR - 1)
    def _():
        pooled = acc_sum[...] / jnp.maximum(acc_cnt[...], 1.0)
        out_ref[...] = (jnp.dot(pooled, lw_ref[...],
                                preferred_element_type=jnp.float32)
                        + lb_ref[...])


_pool = pl.pallas_call(
    _pool_body,
    grid=(GR,),
    in_specs=[
        pl.BlockSpec((NC, BR, F2), lambda i: (0, i, 0)),
        pl.BlockSpec((NC, BR, 1), lambda i: (0, i, 0)),
        pl.BlockSpec((1, BR), lambda i: (0, i)),
        pl.BlockSpec((1, F2), lambda i: (0, 0)),
        pl.BlockSpec((F2, 2), lambda i: (0, 0)),
        pl.BlockSpec((1, 2), lambda i: (0, 0)),
    ],
    out_specs=pl.BlockSpec((G, 2), lambda i: (0, 0)),
    out_shape=jax.ShapeDtypeStruct((G, 2), jnp.float32),
    scratch_shapes=[
        pltpu.VMEM((G, F2), jnp.float32),
        pltpu.VMEM((G, 1), jnp.float32),
    ],
)


def kernel(x, edge_index, batch, W1, att_src1, att_dst1, b1,
           W2, att_src2, att_dst2, b2, lin_w, lin_b):
    f32 = jnp.float32
    loop = jnp.arange(N, dtype=jnp.int32)
    epad = jnp.full((EPAD - ET,), N, jnp.int32)
    srcr = jnp.concatenate([edge_index[0], loop, epad]).reshape(NW, BPT, K)
    dstr = jnp.concatenate([edge_index[1], loop, epad]).reshape(NW, BPT, K)

    eye8 = jnp.eye(8, dtype=f32)
    a_s1 = (att_src1.reshape(H1, 8)[:, :, None] * eye8[:, None, :]).reshape(F1, 8)
    a_d1 = (att_dst1.reshape(H1, 8)[:, :, None] * eye8[:, None, :]).reshape(F1, 8)
    A1 = jnp.concatenate([a_s1, a_d1], axis=1)

    h1, att1 = _dense1(x, W1, A1)

    table1 = jnp.zeros((NTBL, 72), f32)
    table1 = table1.at[:N, :F1].set(h1).at[:N, F1:].set(att1[:, :8])
    adst1_t = jnp.zeros((NTBL, 8), f32).at[:N].set(att1[:, 8:])

    part1 = _edge1()(table1, adst1_t, srcr, dstr)
    S1 = part1[:, :, :F1]
    D1 = part1[:, :, F1:]

    EXP8 = jnp.repeat(eye8, 8, axis=1)
    A2 = jnp.concatenate([att_src2.reshape(F2, 1), att_dst2.reshape(F2, 1)], axis=1)
    h2, att2 = _dense2(S1, D1, b1.reshape(1, F1), EXP8, W2, A2)

    table2 = jnp.zeros((NTBL, F2), f32).at[:N].set(h2)
    asrc2_t = jnp.zeros((NTBL,), f32).at[:N].set(att2[:, 0])
    adst2_t = jnp.zeros((NTBL,), f32).at[:N].set(att2[:, 1])

    part2 = _edge2()(table2, asrc2_t, adst2_t, srcr, dstr)
    S2 = part2[:, :, :F2]
    D2 = part2[:, :, F2:F2 + 1]

    return _pool(S2, D2, batch.reshape(1, N).astype(jnp.int32),
                 b2.reshape(1, F2), lin_w, lin_b.reshape(1, 2))


# speedup vs baseline: 52.3389x; 1.2715x over previous
import functools

import jax
import jax.numpy as jnp
from jax import lax
from jax.experimental import pallas as pl
from jax.experimental.pallas import tpu as pltpu
from jax.experimental.pallas import tpu_sc as plsc

N = 10000
D = 128
F1 = 64
H1 = 8
F2 = 32
G = 128
E = 320000
ET = E + N

NC, NS = 2, 16
NW = NC * NS
K = 128
BPT = -(-ET // (NW * K)) + (-(-ET // (NW * K)) % 2)
EPAD = BPT * NW * K
NPAD = 10240
RPT = NPAD // NS
NTBL = 10016

BR = 128
GR = -(-N // BR)

def _dense1_body(x_ref, w_ref, a_ref, h_ref, att_ref):
    h = jnp.dot(x_ref[...], w_ref[...], preferred_element_type=jnp.float32)
    h_ref[...] = h
    att_ref[...] = jnp.dot(h, a_ref[...], preferred_element_type=jnp.float32)


_dense1 = pl.pallas_call(
    _dense1_body,
    grid=(GR,),
    in_specs=[
        pl.BlockSpec((BR, D), lambda i: (i, 0)),
        pl.BlockSpec((D, F1), lambda i: (0, 0)),
        pl.BlockSpec((F1, 16), lambda i: (0, 0)),
    ],
    out_specs=[
        pl.BlockSpec((BR, F1), lambda i: (i, 0)),
        pl.BlockSpec((BR, 16), lambda i: (i, 0)),
    ],
    out_shape=[
        jax.ShapeDtypeStruct((N, F1), jnp.float32),
        jax.ShapeDtypeStruct((N, 16), jnp.float32),
    ],
)


def _edge1_body(table, adst_t, srcr, dstr, out,
                src_blk, dst_blk, g0, g1, a0, a1, m0, m1, acc,
                sg0, sg1, ss0, ss1):
    cid = lax.axis_index("c")
    sid = lax.axis_index("s")
    wid = sid * NC + cid

    pltpu.sync_copy(srcr.at[wid], src_blk)
    pltpu.sync_copy(dstr.at[wid], dst_blk)

    zero16 = jnp.zeros((16,), jnp.float32)

    def _zrow(i, _):
        for c0 in range(4):
            m0[i, pl.ds(c0 * 16, 16)] = zero16
        m0[i, pl.ds(72 - 16, 16)] = zero16
        return 0
    lax.fori_loop(0, K, _zrow, 0)

    for q in range(RPT // K):
        pltpu.sync_copy(m0, acc.at[pl.ds(sid * RPT + q * K, K)])
    plsc.subcore_barrier()

    lanes = lax.iota(jnp.int32, 16)

    def start_gather(b, gbuf, abuf, sem):
        pltpu.async_copy(table.at[src_blk.at[b]], gbuf, sem)
        pltpu.async_copy(adst_t.at[dst_blk.at[b]], abuf, sem)

    def wait_gather(b, gbuf, abuf, sem):
        pltpu.make_async_copy(table.at[src_blk.at[b]], gbuf, sem).wait()
        pltpu.make_async_copy(adst_t.at[dst_blk.at[b]], abuf, sem).wait()

    def compute(gbuf, abuf, mbuf):
        def _grp(j, _):
            base = j * 16
            ei = lanes + base
            for h in range(H1):
                wcol = jnp.full((16,), F1 + h, jnp.int32)
                asr = plsc.load_gather(gbuf, [ei, wcol])
                ads = plsc.load_gather(abuf, [ei, jnp.full((16,), h, jnp.int32)])
                a = asr + ads
                a = jnp.where(a >= 0.0, a, a * 0.2)
                w = jnp.exp(a)
                plsc.store_scatter(mbuf, [ei, wcol], w)
                for c0 in range(8):
                    fc = jnp.full((16,), h * 8 + c0, jnp.int32)
                    hv = plsc.load_gather(gbuf, [ei, fc])
                    plsc.store_scatter(mbuf, [ei, fc], hv * w)
            return 0
        lax.fori_loop(0, K // 16, _grp, 0)

    start_gather(0, g0, a0, sg0)
    start_gather(1, g1, a1, sg1)

    def _pair(i, _):
        b0 = 2 * i
        b1 = 2 * i + 1
        wait_gather(b0, g0, a0, sg0)

        @pl.when(i > 0)
        def _():
            pltpu.make_async_copy(m0, acc.at[dst_blk.at[b0]], ss0).wait()
        compute(g0, a0, m0)
        pltpu.async_copy(m0, acc.at[dst_blk.at[b0]], ss0, add=True)
        start_gather(jnp.minimum(b0 + 2, BPT - 2), g0, a0, sg0)

        wait_gather(b1, g1, a1, sg1)

        @pl.when(i > 0)
        def _():
            pltpu.make_async_copy(m1, acc.at[dst_blk.at[b1]], ss1).wait()
        compute(g1, a1, m1)
        pltpu.async_copy(m1, acc.at[dst_blk.at[b1]], ss1, add=True)
        start_gather(jnp.minimum(b1 + 2, BPT - 1), g1, a1, sg1)
        return 0
    lax.fori_loop(0, BPT // 2, _pair, 0)

    pltpu.make_async_copy(m0, acc.at[dst_blk.at[0]], ss0).wait()
    pltpu.make_async_copy(m1, acc.at[dst_blk.at[0]], ss1).wait()
    wait_gather(0, g0, a0, sg0)
    wait_gather(0, g1, a1, sg1)

    plsc.subcore_barrier()
    pltpu.sync_copy(acc.at[pl.ds(sid * RPT, RPT)],
                    out.at[cid].at[pl.ds(sid * RPT, RPT)])


@functools.cache
def _edge1():
    mesh = plsc.VectorSubcoreMesh(
        core_axis_name="c", subcore_axis_name="s",
        num_cores=NC, num_subcores=NS)
    return pl.kernel(
        _edge1_body,
        out_type=jax.ShapeDtypeStruct((NC, NPAD, 72), jnp.float32),
        mesh=mesh,
        compiler_params=pltpu.CompilerParams(
            needs_layout_passes=False, use_tc_tiling_on_sc=False),
        scratch_types=[
            pltpu.VMEM((BPT, K), jnp.int32),
            pltpu.VMEM((BPT, K), jnp.int32),
            pltpu.VMEM((K, 72), jnp.float32),
            pltpu.VMEM((K, 72), jnp.float32),
            pltpu.VMEM((K, 8), jnp.float32),
            pltpu.VMEM((K, 8), jnp.float32),
            pltpu.VMEM((K, 72), jnp.float32),
            pltpu.VMEM((K, 72), jnp.float32),
            pltpu.VMEM_SHARED((NPAD, 72), jnp.float32),
            pltpu.SemaphoreType.DMA,
            pltpu.SemaphoreType.DMA,
            pltpu.SemaphoreType.DMA,
            pltpu.SemaphoreType.DMA,
        ],
    )


def _dense2_body(s_ref, d_ref, b1_ref, exp8_ref, w2_ref, a2_ref, h2_ref, att2_ref):
    s = s_ref[0] + s_ref[1]
    den = d_ref[0] + d_ref[1]
    rec = 1.0 / (den + 1e-16)
    recx = jnp.dot(rec, exp8_ref[...], preferred_element_type=jnp.float32)
    h1 = s * recx + b1_ref[...]
    act = jnp.where(h1 > 0.0, h1, jnp.exp(jnp.minimum(h1, 0.0)) - 1.0)
    h2 = jnp.dot(act, w2_ref[...], preferred_element_type=jnp.float32)
    h2_ref[...] = h2
    att2_ref[...] = jnp.dot(h2, a2_ref[...], preferred_element_type=jnp.float32)


_dense2 = pl.pallas_call(
    _dense2_body,
    grid=(GR,),
    in_specs=[
        pl.BlockSpec((NC, BR, F1), lambda i: (0, i, 0)),
        pl.BlockSpec((NC, BR, 8), lambda i: (0, i, 0)),
        pl.BlockSpec((1, F1), lambda i: (0, 0)),
        pl.BlockSpec((8, F1), lambda i: (0, 0)),
        pl.BlockSpec((F1, F2), lambda i: (0, 0)),
        pl.BlockSpec((F2, 2), lambda i: (0, 0)),
    ],
    out_specs=[
        pl.BlockSpec((BR, F2), lambda i: (i, 0)),
        pl.BlockSpec((BR, 2), lambda i: (i, 0)),
    ],
    out_shape=[
        jax.ShapeDtypeStruct((N, F2), jnp.float32),
        jax.ShapeDtypeStruct((N, 2), jnp.float32),
    ],
)


def _edge2_body(table, asrc_t, adst_t, srcr, dstr, out,
                src_blk, dst_blk, g0, g1, m0, m1, asrc_l, adst_l, acc,
                sg0, sg1, ss0, ss1):
    cid = lax.axis_index("c")
    sid = lax.axis_index("s")
    wid = sid * NC + cid

    pltpu.sync_copy(srcr.at[wid], src_blk)
    pltpu.sync_copy(dstr.at[wid], dst_blk)
    pltpu.sync_copy(asrc_t, asrc_l)
    pltpu.sync_copy(adst_t, adst_l)

    zero16 = jnp.zeros((16,), jnp.float32)

    def _zrow(i, _):
        for msg in (m0, m1):
            for c0 in range(2):
                msg[i, pl.ds(c0 * 16, 16)] = zero16
            msg[i, pl.ds(40 - 16, 16)] = zero16
        return 0
    lax.fori_loop(0, K, _zrow, 0)

    for q in range(RPT // K):
        pltpu.sync_copy(m0, acc.at[pl.ds(sid * RPT + q * K, K)])
    plsc.subcore_barrier()

    lanes = lax.iota(jnp.int32, 16)

    def compute(b, gbuf, mbuf):
        def _grp(j, _):
            base = j * 16
            ei = lanes + base
            src16 = src_blk[b, pl.ds(base, 16)]
            dst16 = dst_blk[b, pl.ds(base, 16)]
            asr = plsc.load_gather(asrc_l, [src16])
            ads = plsc.load_gather(adst_l, [dst16])
            a = asr + ads
            a = jnp.where(a >= 0.0, a, a * 0.2)
            w = jnp.exp(a)
            plsc.store_scatter(mbuf, [ei, jnp.full((16,), F2, jnp.int32)], w)
            for c0 in range(F2):
                fc = jnp.full((16,), c0, jnp.int32)
                hv = plsc.load_gather(gbuf, [ei, fc])
                plsc.store_scatter(mbuf, [ei, fc], hv * w)
            return 0
        lax.fori_loop(0, K // 16, _grp, 0)

    pltpu.async_copy(table.at[src_blk.at[0]], g0, sg0)
    pltpu.async_copy(table.at[src_blk.at[1]], g1, sg1)

    def _pair(i, _):
        b0 = 2 * i
        b1 = 2 * i + 1
        pltpu.make_async_copy(table.at[src_blk.at[b0]], g0, sg0).wait()

        @pl.when(i > 0)
        def _():
            pltpu.make_async_copy(m0, acc.at[dst_blk.at[b0]], ss0).wait()
        compute(b0, g0, m0)
        pltpu.async_copy(m0, acc.at[dst_blk.at[b0]], ss0, add=True)
        pltpu.async_copy(
            table.at[src_blk.at[jnp.minimum(b0 + 2, BPT - 2)]], g0, sg0)

        pltpu.make_async_copy(table.at[src_blk.at[b1]], g1, sg1).wait()

        @pl.when(i > 0)
        def _():
            pltpu.make_async_copy(m1, acc.at[dst_blk.at[b1]], ss1).wait()
        compute(b1, g1, m1)
        pltpu.async_copy(m1, acc.at[dst_blk.at[b1]], ss1, add=True)
        pltpu.async_copy(
            table.at[src_blk.at[jnp.minimum(b1 + 2, BPT - 1)]], g1, sg1)
        return 0
    lax.fori_loop(0, BPT // 2, _pair, 0)

    pltpu.make_async_copy(m0, acc.at[dst_blk.at[0]], ss0).wait()
    pltpu.make_async_copy(m1, acc.at[dst_blk.at[0]], ss1).wait()
    pltpu.make_async_copy(table.at[src_blk.at[0]], g0, sg0).wait()
    pltpu.make_async_copy(table.at[src_blk.at[0]], g1, sg1).wait()

    plsc.subcore_barrier()
    pltpu.sync_copy(acc.at[pl.ds(sid * RPT, RPT)],
                    out.at[cid].at[pl.ds(sid * RPT, RPT)])


@functools.cache
def _edge2():
    mesh = plsc.VectorSubcoreMesh(
        core_axis_name="c", subcore_axis_name="s",
        num_cores=NC, num_subcores=NS)
    return pl.kernel(
        _edge2_body,
        out_type=jax.ShapeDtypeStruct((NC, NPAD, 40), jnp.float32),
        mesh=mesh,
        compiler_params=pltpu.CompilerParams(
            needs_layout_passes=False, use_tc_tiling_on_sc=False),
        scratch_types=[
            pltpu.VMEM((BPT, K), jnp.int32),
            pltpu.VMEM((BPT, K), jnp.int32),
            pltpu.VMEM((K, F2), jnp.float32),
            pltpu.VMEM((K, F2), jnp.float32),
            pltpu.VMEM((K, 40), jnp.float32),
            pltpu.VMEM((K, 40), jnp.float32),
            pltpu.VMEM((NTBL,), jnp.float32),
            pltpu.VMEM((NTBL,), jnp.float32),
            pltpu.VMEM_SHARED((NPAD, 40), jnp.float32),
            pltpu.SemaphoreType.DMA,
            pltpu.SemaphoreType.DMA,
            pltpu.SemaphoreType.DMA,
            pltpu.SemaphoreType.DMA,
        ],
    )


def _pool_body(s2_ref, d2_ref, bt_ref, b2_ref, lw_ref, lb_ref, out_ref,
               acc_sum, acc_cnt):
    i = pl.program_id(0)

    @pl.when(i == 0)
    def _():
        acc_sum[...] = jnp.zeros((G, F2), jnp.float32)
        acc_cnt[...] = jnp.zeros((G, 1), jnp.float32)

    s = s2_ref[0] + s2_ref[1]
    den = d2_ref[0] + d2_ref[1]
    h = s / (den + 1e-16) + b2_ref[...]
    colidx = i * BR + lax.broadcasted_iota(jnp.int32, (1, BR), 1)
    validt = colidx < N
    gids = lax.broadcasted_iota(jnp.int32, (G, BR), 0)
    oht = jnp.where((bt_ref[...] == gids) & validt, 1.0, 0.0)
    acc_sum[...] += jnp.dot(oht, h, preferred_element_type=jnp.float32)
    acc_cnt[...] += jnp.sum(oht, axis=1, keepdims=True)

    @pl.when(i == GR - 1)
    def _():
        pooled = acc_sum[...] / jnp.maximum(acc_cnt[...], 1.0)
        out_ref[...] = (jnp.dot(pooled, lw_ref[...],
                                preferred_element_type=jnp.float32)
                        + lb_ref[...])


_pool = pl.pallas_call(
    _pool_body,
    grid=(GR,),
    in_specs=[
        pl.BlockSpec((NC, BR, F2), lambda i: (0, i, 0)),
        pl.BlockSpec((NC, BR, 1), lambda i: (0, i, 0)),
        pl.BlockSpec((1, BR), lambda i: (0, i)),
        pl.BlockSpec((1, F2), lambda i: (0, 0)),
        pl.BlockSpec((F2, 2), lambda i: (0, 0)),
        pl.BlockSpec((1, 2), lambda i: (0, 0)),
    ],
    out_specs=pl.BlockSpec((G, 2), lambda i: (0, 0)),
    out_shape=jax.ShapeDtypeStruct((G, 2), jnp.float32),
    scratch_shapes=[
        pltpu.VMEM((G, F2), jnp.float32),
        pltpu.VMEM((G, 1), jnp.float32),
    ],
)


def kernel(x, edge_index, batch, W1, att_src1, att_dst1, b1,
           W2, att_src2, att_dst2, b2, lin_w, lin_b):
    f32 = jnp.float32
    loop = jnp.arange(N, dtype=jnp.int32)
    epad = jnp.full((EPAD - ET,), N, jnp.int32)
    srcr = jnp.concatenate([edge_index[0], loop, epad]).reshape(NW, BPT, K)
    dstr = jnp.concatenate([edge_index[1], loop, epad]).reshape(NW, BPT, K)

    eye8 = jnp.eye(8, dtype=f32)
    a_s1 = (att_src1.reshape(H1, 8)[:, :, None] * eye8[:, None, :]).reshape(F1, 8)
    a_d1 = (att_dst1.reshape(H1, 8)[:, :, None] * eye8[:, None, :]).reshape(F1, 8)
    A1 = jnp.concatenate([a_s1, a_d1], axis=1)

    h1, att1 = _dense1(x, W1, A1)

    table1 = jnp.zeros((NTBL, 72), f32)
    table1 = table1.at[:N, :F1].set(h1).at[:N, F1:].set(att1[:, :8])
    adst1_t = jnp.zeros((NTBL, 8), f32).at[:N].set(att1[:, 8:])

    part1 = _edge1()(table1, adst1_t, srcr, dstr)
    S1 = part1[:, :, :F1]
    D1 = part1[:, :, F1:]

    EXP8 = jnp.repeat(eye8, 8, axis=1)
    A2 = jnp.concatenate([att_src2.reshape(F2, 1), att_dst2.reshape(F2, 1)], axis=1)
    h2, att2 = _dense2(S1, D1, b1.reshape(1, F1), EXP8, W2, A2)

    table2 = jnp.zeros((NTBL, F2), f32).at[:N].set(h2)
    asrc2_t = jnp.zeros((NTBL,), f32).at[:N].set(att2[:, 0])
    adst2_t = jnp.zeros((NTBL,), f32).at[:N].set(att2[:, 1])

    part2 = _edge2()(table2, asrc2_t, adst2_t, srcr, dstr)
    S2 = part2[:, :, :F2]
    D2 = part2[:, :, F2:F2 + 1]

    return _pool(S2, D2, batch.reshape(1, N).astype(jnp.int32),
                 b2.reshape(1, F2), lin_w, lin_b.reshape(1, 2))


# speedup vs baseline: 54.2834x; 1.0372x over previous
import functools

import jax
import jax.numpy as jnp
from jax import lax
from jax.experimental import pallas as pl
from jax.experimental.pallas import tpu as pltpu
from jax.experimental.pallas import tpu_sc as plsc

N = 10000
D = 128
F1 = 64
H1 = 8
F2 = 32
G = 128
E = 320000
ET = E + N

NC, NS = 2, 16
NW = NC * NS
K = 128
BPT = -(-ET // (NW * K)) + (-(-ET // (NW * K)) % 2)
EPAD = BPT * NW * K
NPAD = 10240
RPT = NPAD // NS
NTBL = 10016

BR = 128
GR = -(-N // BR)

def _dense1_body(x_ref, w_ref, a_ref, t_ref, ad_ref):
    h = jnp.dot(x_ref[...], w_ref[...], preferred_element_type=jnp.float32)
    att = jnp.dot(h, a_ref[...], preferred_element_type=jnp.float32)
    rows = pl.program_id(0) * BR + lax.broadcasted_iota(jnp.int32, (BR, 1), 0)
    valid = rows < N
    t = jnp.concatenate([h, att[:, :H1]], axis=1)
    t_ref[...] = jnp.where(valid, t, 0.0)
    ad_ref[...] = jnp.where(valid, att[:, H1:], 0.0)


_dense1 = pl.pallas_call(
    _dense1_body,
    grid=(GR,),
    in_specs=[
        pl.BlockSpec((BR, D), lambda i: (i, 0)),
        pl.BlockSpec((D, F1), lambda i: (0, 0)),
        pl.BlockSpec((F1, 16), lambda i: (0, 0)),
    ],
    out_specs=[
        pl.BlockSpec((BR, 72), lambda i: (i, 0)),
        pl.BlockSpec((BR, 8), lambda i: (i, 0)),
    ],
    out_shape=[
        jax.ShapeDtypeStruct((NTBL, 72), jnp.float32),
        jax.ShapeDtypeStruct((NTBL, 8), jnp.float32),
    ],
)


def _edge1_body(table, adst_t, srcr, dstr, out,
                src_blk, dst_blk, g0, g1, a0, a1, m0, m1, acc,
                sg0, sg1, ss0, ss1):
    cid = lax.axis_index("c")
    sid = lax.axis_index("s")
    wid = sid * NC + cid

    pltpu.sync_copy(srcr.at[wid], src_blk)
    pltpu.sync_copy(dstr.at[wid], dst_blk)

    zero16 = jnp.zeros((16,), jnp.float32)

    def _zrow(i, _):
        for c0 in range(4):
            m0[i, pl.ds(c0 * 16, 16)] = zero16
        m0[i, pl.ds(72 - 16, 16)] = zero16
        return 0
    lax.fori_loop(0, K, _zrow, 0)

    for q in range(RPT // K):
        pltpu.sync_copy(m0, acc.at[pl.ds(sid * RPT + q * K, K)])
    plsc.subcore_barrier()

    lanes = lax.iota(jnp.int32, 16)

    def start_gather(b, gbuf, abuf, sem):
        pltpu.async_copy(table.at[src_blk.at[b]], gbuf, sem)
        pltpu.async_copy(adst_t.at[dst_blk.at[b]], abuf, sem)

    def wait_gather(b, gbuf, abuf, sem):
        pltpu.make_async_copy(table.at[src_blk.at[b]], gbuf, sem).wait()
        pltpu.make_async_copy(adst_t.at[dst_blk.at[b]], abuf, sem).wait()

    def compute(gbuf, abuf, mbuf):
        def _grp(j, _):
            base = j * 16
            ei = lanes + base
            for h in range(H1):
                wcol = jnp.full((16,), F1 + h, jnp.int32)
                asr = plsc.load_gather(gbuf, [ei, wcol])
                ads = plsc.load_gather(abuf, [ei, jnp.full((16,), h, jnp.int32)])
                a = asr + ads
                a = jnp.where(a >= 0.0, a, a * 0.2)
                w = jnp.exp(a)
                plsc.store_scatter(mbuf, [ei, wcol], w)
                for c0 in range(8):
                    fc = jnp.full((16,), h * 8 + c0, jnp.int32)
                    hv = plsc.load_gather(gbuf, [ei, fc])
                    plsc.store_scatter(mbuf, [ei, fc], hv * w)
            return 0
        lax.fori_loop(0, K // 16, _grp, 0)

    start_gather(0, g0, a0, sg0)
    start_gather(1, g1, a1, sg1)

    def _pair(i, _):
        b0 = 2 * i
        b1 = 2 * i + 1
        wait_gather(b0, g0, a0, sg0)

        @pl.when(i > 0)
        def _():
            pltpu.make_async_copy(m0, acc.at[dst_blk.at[b0]], ss0).wait()
        compute(g0, a0, m0)
        pltpu.async_copy(m0, acc.at[dst_blk.at[b0]], ss0, add=True)
        start_gather(jnp.minimum(b0 + 2, BPT - 2), g0, a0, sg0)

        wait_gather(b1, g1, a1, sg1)

        @pl.when(i > 0)
        def _():
            pltpu.make_async_copy(m1, acc.at[dst_blk.at[b1]], ss1).wait()
        compute(g1, a1, m1)
        pltpu.async_copy(m1, acc.at[dst_blk.at[b1]], ss1, add=True)
        start_gather(jnp.minimum(b1 + 2, BPT - 1), g1, a1, sg1)
        return 0
    lax.fori_loop(0, BPT // 2, _pair, 0)

    pltpu.make_async_copy(m0, acc.at[dst_blk.at[0]], ss0).wait()
    pltpu.make_async_copy(m1, acc.at[dst_blk.at[0]], ss1).wait()
    wait_gather(0, g0, a0, sg0)
    wait_gather(0, g1, a1, sg1)

    plsc.subcore_barrier()
    pltpu.sync_copy(acc.at[pl.ds(sid * RPT, RPT)],
                    out.at[cid].at[pl.ds(sid * RPT, RPT)])


@functools.cache
def _edge1():
    mesh = plsc.VectorSubcoreMesh(
        core_axis_name="c", subcore_axis_name="s",
        num_cores=NC, num_subcores=NS)
    return pl.kernel(
        _edge1_body,
        out_type=jax.ShapeDtypeStruct((NC, NPAD, 72), jnp.float32),
        mesh=mesh,
        compiler_params=pltpu.CompilerParams(
            needs_layout_passes=False, use_tc_tiling_on_sc=False),
        scratch_types=[
            pltpu.VMEM((BPT, K), jnp.int32),
            pltpu.VMEM((BPT, K), jnp.int32),
            pltpu.VMEM((K, 72), jnp.float32),
            pltpu.VMEM((K, 72), jnp.float32),
            pltpu.VMEM((K, 8), jnp.float32),
            pltpu.VMEM((K, 8), jnp.float32),
            pltpu.VMEM((K, 72), jnp.float32),
            pltpu.VMEM((K, 72), jnp.float32),
            pltpu.VMEM_SHARED((NPAD, 72), jnp.float32),
            pltpu.SemaphoreType.DMA,
            pltpu.SemaphoreType.DMA,
            pltpu.SemaphoreType.DMA,
            pltpu.SemaphoreType.DMA,
        ],
    )


def _dense2_body(p_ref, b1_ref, exp8_ref, w2_ref, a2_ref, t_ref, at_ref):
    s = p_ref[0, :, :F1] + p_ref[1, :, :F1]
    den = p_ref[0, :, F1:] + p_ref[1, :, F1:]
    rec = 1.0 / (den + 1e-16)
    recx = jnp.dot(rec, exp8_ref[...], preferred_element_type=jnp.float32)
    h1 = s * recx + b1_ref[...]
    act = jnp.where(h1 > 0.0, h1, jnp.exp(jnp.minimum(h1, 0.0)) - 1.0)
    h2 = jnp.dot(act, w2_ref[...], preferred_element_type=jnp.float32)
    att2 = jnp.dot(h2, a2_ref[...], preferred_element_type=jnp.float32)
    rows = pl.program_id(0) * BR + lax.broadcasted_iota(jnp.int32, (BR, 1), 0)
    valid = rows < N
    t_ref[...] = jnp.where(valid, h2, 0.0)
    at_ref[...] = jnp.where(valid, att2, 0.0)


_dense2 = pl.pallas_call(
    _dense2_body,
    grid=(GR,),
    in_specs=[
        pl.BlockSpec((NC, BR, 72), lambda i: (0, i, 0)),
        pl.BlockSpec((1, F1), lambda i: (0, 0)),
        pl.BlockSpec((8, F1), lambda i: (0, 0)),
        pl.BlockSpec((F1, F2), lambda i: (0, 0)),
        pl.BlockSpec((F2, 2), lambda i: (0, 0)),
    ],
    out_specs=[
        pl.BlockSpec((BR, F2), lambda i: (i, 0)),
        pl.BlockSpec((BR, 2), lambda i: (i, 0)),
    ],
    out_shape=[
        jax.ShapeDtypeStruct((NTBL, F2), jnp.float32),
        jax.ShapeDtypeStruct((NTBL, 2), jnp.float32),
    ],
)


def _edge2_body(table, asrc_t, adst_t, srcr, dstr, out,
                src_blk, dst_blk, g0, g1, m0, m1, asrc_l, adst_l, acc,
                sg0, sg1, ss0, ss1):
    cid = lax.axis_index("c")
    sid = lax.axis_index("s")
    wid = sid * NC + cid

    pltpu.sync_copy(srcr.at[wid], src_blk)
    pltpu.sync_copy(dstr.at[wid], dst_blk)
    pltpu.sync_copy(asrc_t, asrc_l)
    pltpu.sync_copy(adst_t, adst_l)

    zero16 = jnp.zeros((16,), jnp.float32)

    def _zrow(i, _):
        for msg in (m0, m1):
            for c0 in range(2):
                msg[i, pl.ds(c0 * 16, 16)] = zero16
            msg[i, pl.ds(40 - 16, 16)] = zero16
        return 0
    lax.fori_loop(0, K, _zrow, 0)

    for q in range(RPT // K):
        pltpu.sync_copy(m0, acc.at[pl.ds(sid * RPT + q * K, K)])
    plsc.subcore_barrier()

    lanes = lax.iota(jnp.int32, 16)

    def compute(b, gbuf, mbuf):
        def _grp(j, _):
            base = j * 16
            ei = lanes + base
            src16 = src_blk[b, pl.ds(base, 16)]
            dst16 = dst_blk[b, pl.ds(base, 16)]
            asr = plsc.load_gather(asrc_l, [src16])
            ads = plsc.load_gather(adst_l, [dst16])
            a = asr + ads
            a = jnp.where(a >= 0.0, a, a * 0.2)
            w = jnp.exp(a)
            plsc.store_scatter(mbuf, [ei, jnp.full((16,), F2, jnp.int32)], w)
            for c0 in range(F2):
                fc = jnp.full((16,), c0, jnp.int32)
                hv = plsc.load_gather(gbuf, [ei, fc])
                plsc.store_scatter(mbuf, [ei, fc], hv * w)
            return 0
        lax.fori_loop(0, K // 16, _grp, 0)

    pltpu.async_copy(table.at[src_blk.at[0]], g0, sg0)
    pltpu.async_copy(table.at[src_blk.at[1]], g1, sg1)

    def _pair(i, _):
        b0 = 2 * i
        b1 = 2 * i + 1
        pltpu.make_async_copy(table.at[src_blk.at[b0]], g0, sg0).wait()

        @pl.when(i > 0)
        def _():
            pltpu.make_async_copy(m0, acc.at[dst_blk.at[b0]], ss0).wait()
        compute(b0, g0, m0)
        pltpu.async_copy(m0, acc.at[dst_blk.at[b0]], ss0, add=True)
        pltpu.async_copy(
            table.at[src_blk.at[jnp.minimum(b0 + 2, BPT - 2)]], g0, sg0)

        pltpu.make_async_copy(table.at[src_blk.at[b1]], g1, sg1).wait()

        @pl.when(i > 0)
        def _():
            pltpu.make_async_copy(m1, acc.at[dst_blk.at[b1]], ss1).wait()
        compute(b1, g1, m1)
        pltpu.async_copy(m1, acc.at[dst_blk.at[b1]], ss1, add=True)
        pltpu.async_copy(
            table.at[src_blk.at[jnp.minimum(b1 + 2, BPT - 1)]], g1, sg1)
        return 0
    lax.fori_loop(0, BPT // 2, _pair, 0)

    pltpu.make_async_copy(m0, acc.at[dst_blk.at[0]], ss0).wait()
    pltpu.make_async_copy(m1, acc.at[dst_blk.at[0]], ss1).wait()
    pltpu.make_async_copy(table.at[src_blk.at[0]], g0, sg0).wait()
    pltpu.make_async_copy(table.at[src_blk.at[0]], g1, sg1).wait()

    plsc.subcore_barrier()
    pltpu.sync_copy(acc.at[pl.ds(sid * RPT, RPT)],
                    out.at[cid].at[pl.ds(sid * RPT, RPT)])


@functools.cache
def _edge2():
    mesh = plsc.VectorSubcoreMesh(
        core_axis_name="c", subcore_axis_name="s",
        num_cores=NC, num_subcores=NS)
    return pl.kernel(
        _edge2_body,
        out_type=jax.ShapeDtypeStruct((NC, NPAD, 40), jnp.float32),
        mesh=mesh,
        compiler_params=pltpu.CompilerParams(
            needs_layout_passes=False, use_tc_tiling_on_sc=False),
        scratch_types=[
            pltpu.VMEM((BPT, K), jnp.int32),
            pltpu.VMEM((BPT, K), jnp.int32),
            pltpu.VMEM((K, F2), jnp.float32),
            pltpu.VMEM((K, F2), jnp.float32),
            pltpu.VMEM((K, 40), jnp.float32),
            pltpu.VMEM((K, 40), jnp.float32),
            pltpu.VMEM((NTBL,), jnp.float32),
            pltpu.VMEM((NTBL,), jnp.float32),
            pltpu.VMEM_SHARED((NPAD, 40), jnp.float32),
            pltpu.SemaphoreType.DMA,
            pltpu.SemaphoreType.DMA,
            pltpu.SemaphoreType.DMA,
            pltpu.SemaphoreType.DMA,
        ],
    )


def _pool_body(p_ref, bt_ref, b2_ref, lw_ref, lb_ref, out_ref,
               acc_sum, acc_cnt):
    i = pl.program_id(0)

    @pl.when(i == 0)
    def _():
        acc_sum[...] = jnp.zeros((G, F2), jnp.float32)
        acc_cnt[...] = jnp.zeros((G, 1), jnp.float32)

    s = p_ref[0, :, :F2] + p_ref[1, :, :F2]
    den = p_ref[0, :, F2:F2 + 1] + p_ref[1, :, F2:F2 + 1]
    h = s / (den + 1e-16) + b2_ref[...]
    colidx = i * BR + lax.broadcasted_iota(jnp.int32, (1, BR), 1)
    validt = colidx < N
    gids = lax.broadcasted_iota(jnp.int32, (G, BR), 0)
    oht = jnp.where((bt_ref[...] == gids) & validt, 1.0, 0.0)
    acc_sum[...] += jnp.dot(oht, h, preferred_element_type=jnp.float32)
    acc_cnt[...] += jnp.sum(oht, axis=1, keepdims=True)

    @pl.when(i == GR - 1)
    def _():
        pooled = acc_sum[...] / jnp.maximum(acc_cnt[...], 1.0)
        out_ref[...] = (jnp.dot(pooled, lw_ref[...],
                                preferred_element_type=jnp.float32)
                        + lb_ref[...])


_pool = pl.pallas_call(
    _pool_body,
    grid=(GR,),
    in_specs=[
        pl.BlockSpec((NC, BR, 40), lambda i: (0, i, 0)),
        pl.BlockSpec((1, BR), lambda i: (0, i)),
        pl.BlockSpec((1, F2), lambda i: (0, 0)),
        pl.BlockSpec((F2, 2), lambda i: (0, 0)),
        pl.BlockSpec((1, 2), lambda i: (0, 0)),
    ],
    out_specs=pl.BlockSpec((G, 2), lambda i: (0, 0)),
    out_shape=jax.ShapeDtypeStruct((G, 2), jnp.float32),
    scratch_shapes=[
        pltpu.VMEM((G, F2), jnp.float32),
        pltpu.VMEM((G, 1), jnp.float32),
    ],
)


def kernel(x, edge_index, batch, W1, att_src1, att_dst1, b1,
           W2, att_src2, att_dst2, b2, lin_w, lin_b):
    loop = jnp.arange(N, dtype=jnp.int32)
    epad = jnp.full((EPAD - ET,), N, jnp.int32)
    srcr = jnp.concatenate([edge_index[0], loop, epad]).reshape(NW, BPT, K)
    dstr = jnp.concatenate([edge_index[1], loop, epad]).reshape(NW, BPT, K)

    eye8 = jnp.eye(8, dtype=jnp.float32)
    a_s1 = (att_src1.reshape(H1, 8)[:, :, None] * eye8[:, None, :]).reshape(F1, 8)
    a_d1 = (att_dst1.reshape(H1, 8)[:, :, None] * eye8[:, None, :]).reshape(F1, 8)
    A1 = jnp.concatenate([a_s1, a_d1], axis=1)

    table1, adst1_t = _dense1(x, W1, A1)
    part1 = _edge1()(table1, adst1_t, srcr, dstr)

    EXP8 = jnp.repeat(eye8, 8, axis=1)
    A2 = jnp.concatenate([att_src2.reshape(F2, 1), att_dst2.reshape(F2, 1)], axis=1)
    table2, att2 = _dense2(part1, b1.reshape(1, F1), EXP8, W2, A2)

    part2 = _edge2()(table2, att2[:, 0], att2[:, 1], srcr, dstr)

    return _pool(part2, batch.reshape(1, N).astype(jnp.int32),
                 b2.reshape(1, F2), lin_w, lin_b.reshape(1, 2))
